# baseline (speedup 1.0000x reference)
import functools

import jax
import jax.numpy as jnp
from jax.experimental import pallas as pl
from jax.experimental.pallas import tpu as pltpu

P = 128
H = 32
E = 1024


def _graph_kernel(src_ref, dst_ref, W_enc_ref, b_enc_ref, W_g_ref, b_g_ref,
                  A_ref, Wc_ref, bias_ref):
    src = src_ref[...]
    dst = dst_ref[...]
    node = jax.lax.broadcasted_iota(jnp.int32, (E, P), 1)
    U = (src == node).astype(jnp.float32)
    V = (dst == node).astype(jnp.float32)
    cnt = jax.lax.dot_general(V, U, (((0,), (0,)), ((), ())))
    out_deg = jnp.sum(U, axis=0) + 1.0
    in_deg = jnp.sum(V, axis=0) + 1.0
    eye = (jax.lax.broadcasted_iota(jnp.int32, (P, P), 0) ==
           jax.lax.broadcasted_iota(jnp.int32, (P, P), 1)).astype(jnp.float32)
    An = (jax.lax.rsqrt(in_deg)[:, None] * (cnt + eye) *
          jax.lax.rsqrt(out_deg)[None, :])
    A_ref[...] = An
    Wc = jnp.dot(W_enc_ref[...], W_g_ref[...])
    Wc_ref[...] = Wc
    c1 = jnp.dot(b_enc_ref[...], W_g_ref[...])
    bias_ref[...] = jnp.sum(An, axis=1)[:, None] * c1 + b_g_ref[...]


def _main_kernel(x_ref, A_ref, Wc_ref, bias_ref, out_ref, *, bb):
    xb = x_ref[...]
    x3 = xb.T.reshape(P, H, bb)
    u = jax.lax.dot_general(
        A_ref[...], x3, (((1,), (0,)), ((), ())))
    u = jnp.transpose(u, (0, 2, 1))
    w = jnp.dot(u.reshape(P * bb, H), Wc_ref[...])
    w = jnp.transpose(w.reshape(P, bb, H), (1, 0, 2))
    out_ref[...] = (w + bias_ref[...][None, :, :]).reshape(bb * P, H)


def kernel(x, W_enc, b_enc, W_g, b_g, src, dst):
    B = x.shape[0]
    A, Wc, bias = pl.pallas_call(
        _graph_kernel,
        out_shape=(
            jax.ShapeDtypeStruct((P, P), jnp.float32),
            jax.ShapeDtypeStruct((H, H), jnp.float32),
            jax.ShapeDtypeStruct((P, H), jnp.float32),
        ),
    )(src.reshape(E, 1), dst.reshape(E, 1),
      W_enc, b_enc.reshape(1, H), W_g, b_g.reshape(1, H))

    bb = 64
    out = pl.pallas_call(
        functools.partial(_main_kernel, bb=bb),
        grid=(B // bb,),
        in_specs=[
            pl.BlockSpec((bb, P * H), lambda i: (i, 0)),
            pl.BlockSpec((P, P), lambda i: (0, 0)),
            pl.BlockSpec((H, H), lambda i: (0, 0)),
            pl.BlockSpec((P, H), lambda i: (0, 0)),
        ],
        out_specs=pl.BlockSpec((bb * P, H), lambda i: (i, 0)),
        out_shape=jax.ShapeDtypeStruct((B * P, H), jnp.float32),
        compiler_params=pltpu.CompilerParams(
            dimension_semantics=("parallel",)),
    )(x, A, Wc, bias)
    return out


# speedup vs baseline: 1.0528x; 1.0528x over previous
import functools

import jax
import jax.numpy as jnp
from jax.experimental import pallas as pl
from jax.experimental.pallas import tpu as pltpu

P = 128
H = 32
E = 1024


def _graph_kernel(src_ref, dst_ref, W_enc_ref, b_enc_ref, W_g_ref, b_g_ref,
                  A_ref, Wc_ref, bias_ref):
    src = src_ref[...]
    dst = dst_ref[...]
    node = jax.lax.broadcasted_iota(jnp.int32, (E, P), 1)
    U = (src == node).astype(jnp.float32)
    V = (dst == node).astype(jnp.float32)
    cnt = jax.lax.dot_general(V, U, (((0,), (0,)), ((), ())))
    out_deg = jnp.sum(U, axis=0) + 1.0
    in_deg = jnp.sum(V, axis=0) + 1.0
    eye = (jax.lax.broadcasted_iota(jnp.int32, (P, P), 0) ==
           jax.lax.broadcasted_iota(jnp.int32, (P, P), 1)).astype(jnp.float32)
    An = (jax.lax.rsqrt(in_deg)[:, None] * (cnt + eye) *
          jax.lax.rsqrt(out_deg)[None, :])
    A_ref[...] = An
    Wc = jnp.dot(W_enc_ref[...], W_g_ref[...])
    Wc_ref[...] = Wc
    c1 = jnp.dot(b_enc_ref[...], W_g_ref[...])
    bias_ref[...] = jnp.sum(An, axis=1)[:, None] * c1 + b_g_ref[...]


def _main_kernel(x_ref, A_ref, Wc_ref, bias_ref, out_ref, *, bb):
    xb = x_ref[...]
    x3 = xb.T.reshape(P, H, bb)
    u = jax.lax.dot_general(
        A_ref[...], x3, (((1,), (0,)), ((), ())))
    u = jnp.transpose(u, (0, 2, 1))
    w = jnp.dot(u.reshape(P * bb, H), Wc_ref[...])
    w = jnp.transpose(w.reshape(P, bb, H), (1, 0, 2))
    out_ref[...] = (w + bias_ref[...][None, :, :]).reshape(bb * P, H)


def kernel(x, W_enc, b_enc, W_g, b_g, src, dst):
    B = x.shape[0]
    A, Wc, bias = pl.pallas_call(
        _graph_kernel,
        out_shape=(
            jax.ShapeDtypeStruct((P, P), jnp.float32),
            jax.ShapeDtypeStruct((H, H), jnp.float32),
            jax.ShapeDtypeStruct((P, H), jnp.float32),
        ),
    )(src.reshape(E, 1), dst.reshape(E, 1),
      W_enc, b_enc.reshape(1, H), W_g, b_g.reshape(1, H))

    bb = 256
    out = pl.pallas_call(
        functools.partial(_main_kernel, bb=bb),
        grid=(B // bb,),
        in_specs=[
            pl.BlockSpec((bb, P * H), lambda i: (i, 0)),
            pl.BlockSpec((P, P), lambda i: (0, 0)),
            pl.BlockSpec((H, H), lambda i: (0, 0)),
            pl.BlockSpec((P, H), lambda i: (0, 0)),
        ],
        out_specs=pl.BlockSpec((bb * P, H), lambda i: (i, 0)),
        out_shape=jax.ShapeDtypeStruct((B * P, H), jnp.float32),
        compiler_params=pltpu.CompilerParams(
            dimension_semantics=("parallel",)),
    )(x, A, Wc, bias)
    return out


# speedup vs baseline: 1.1390x; 1.0819x over previous
import functools

import jax
import jax.numpy as jnp
from jax.experimental import pallas as pl
from jax.experimental.pallas import tpu as pltpu

P = 128
H = 32
E = 1024


def _fused_kernel(src_ref, dst_ref, W_enc_ref, b_enc_ref, W_g_ref, b_g_ref,
                  x_ref, out_ref, A_ref, Wc_ref, bias_ref, *, bb):
    @pl.when(pl.program_id(0) == 0)
    def _build_graph():
        src = src_ref[...]
        dst = dst_ref[...]
        node = jax.lax.broadcasted_iota(jnp.int32, (E, P), 1)
        U = (src == node).astype(jnp.float32)
        V = (dst == node).astype(jnp.float32)
        cnt = jax.lax.dot_general(V, U, (((0,), (0,)), ((), ())))
        out_deg = jnp.sum(U, axis=0) + 1.0
        in_deg = jnp.sum(V, axis=0) + 1.0
        eye = (jax.lax.broadcasted_iota(jnp.int32, (P, P), 0) ==
               jax.lax.broadcasted_iota(jnp.int32, (P, P), 1)
               ).astype(jnp.float32)
        An = (jax.lax.rsqrt(in_deg)[:, None] * (cnt + eye) *
              jax.lax.rsqrt(out_deg)[None, :])
        A_ref[...] = An
        Wc_ref[...] = jnp.dot(W_enc_ref[...], W_g_ref[...])
        c1 = jnp.dot(b_enc_ref[...], W_g_ref[...])
        bias_ref[...] = jnp.sum(An, axis=1)[:, None] * c1 + b_g_ref[...]

    xb = x_ref[...]
    x3 = xb.T.reshape(P, H, bb)
    u = jax.lax.dot_general(
        A_ref[...], x3, (((1,), (0,)), ((), ())))
    u = jnp.transpose(u, (0, 2, 1))
    w = jnp.dot(u.reshape(P * bb, H), Wc_ref[...])
    w = jnp.transpose(w.reshape(P, bb, H), (1, 0, 2))
    out_ref[...] = (w + bias_ref[...][None, :, :]).reshape(bb * P, H)


def kernel(x, W_enc, b_enc, W_g, b_g, src, dst):
    B = x.shape[0]
    bb = 128
    out = pl.pallas_call(
        functools.partial(_fused_kernel, bb=bb),
        grid=(B // bb,),
        in_specs=[
            pl.BlockSpec((E, 1), lambda i: (0, 0)),
            pl.BlockSpec((E, 1), lambda i: (0, 0)),
            pl.BlockSpec((H, H), lambda i: (0, 0)),
            pl.BlockSpec((1, H), lambda i: (0, 0)),
            pl.BlockSpec((H, H), lambda i: (0, 0)),
            pl.BlockSpec((1, H), lambda i: (0, 0)),
            pl.BlockSpec((bb, P * H), lambda i: (i, 0)),
        ],
        out_specs=pl.BlockSpec((bb * P, H), lambda i: (i, 0)),
        out_shape=jax.ShapeDtypeStruct((B * P, H), jnp.float32),
        scratch_shapes=[
            pltpu.VMEM((P, P), jnp.float32),
            pltpu.VMEM((H, H), jnp.float32),
            pltpu.VMEM((P, H), jnp.float32),
        ],
    )(src.reshape(E, 1), dst.reshape(E, 1), W_enc, b_enc.reshape(1, H),
      W_g, b_g.reshape(1, H), x)
    return out
